# baseline (speedup 1.0000x reference)
import jax
import jax.numpy as jnp
from jax import lax
from jax.experimental import pallas as pl
from jax.experimental.pallas import tpu as pltpu

_BETA = 0.25
_BB = 128


def _vq_body(e_ref, esq_ref, wt_ref, wsq_ref, w_ref,
             idx_ref, st_ref, loss_ref):
    e = e_ref[...]
    mm = jnp.dot(e, wt_ref[...],
                 preferred_element_type=jnp.float32)
    d2 = (esq_ref[...] - 2.0 * mm) + wsq_ref[...]
    k = d2.shape[1]
    h = k // 2
    iota = lax.broadcasted_iota(jnp.int32, d2.shape, 1)
    dl, dr = d2[:, :h], d2[:, h:]
    il_iota, ir_iota = iota[:, :h], iota[:, h:]
    vl = jnp.min(dl, axis=1, keepdims=True)
    vr = jnp.min(dr, axis=1, keepdims=True)
    il = jnp.min(jnp.where(dl == vl, il_iota, k), axis=1)
    ir = jnp.min(jnp.where(dr == vr, ir_iota, k), axis=1)
    vl_bf = vl.astype(jnp.bfloat16).astype(jnp.float32)
    pickr = vr < vl_bf
    idx = jnp.where(pickr[:, 0], ir, il)
    idx_ref[0, 0, :] = idx
    oh = (iota == idx[:, None]).astype(jnp.float32)
    q = jnp.dot(oh, w_ref[...], preferred_element_type=jnp.float32)
    st_ref[...] = e + (q - e)
    dsel = jnp.where(pickr[:, 0], vr[:, 0], vl[:, 0])
    @pl.when(pl.program_id(0) == 0)
    def _():
        loss_ref[...] = jnp.zeros((1, 1), jnp.float32)
    loss_ref[...] += jnp.sum(dsel).reshape(1, 1)


def kernel(encoding, embedding_weight):
    b, d = encoding.shape
    k = embedding_weight.shape[0]
    enc_sq = jnp.sum(encoding * encoding, axis=1, keepdims=True)
    w_sq = jnp.sum(embedding_weight * embedding_weight, axis=1)
    wt = embedding_weight.T
    g = b // _BB

    idx3, st, loss = pl.pallas_call(
        _vq_body,
        grid=(g,),
        in_specs=[
            pl.BlockSpec((_BB, d), lambda i: (i, 0)),
            pl.BlockSpec((_BB, 1), lambda i: (i, 0)),
            pl.BlockSpec((d, k), lambda i: (0, 0)),
            pl.BlockSpec((1, k), lambda i: (0, 0)),
            pl.BlockSpec((k, d), lambda i: (0, 0)),
        ],
        out_specs=[
            pl.BlockSpec((1, 1, _BB), lambda i: (i, 0, 0)),
            pl.BlockSpec((_BB, d), lambda i: (i, 0)),
            pl.BlockSpec((1, 1), lambda i: (0, 0)),
        ],
        out_shape=[
            jax.ShapeDtypeStruct((g, 1, _BB), jnp.int32),
            jax.ShapeDtypeStruct((b, d), jnp.float32),
            jax.ShapeDtypeStruct((1, 1), jnp.float32),
        ],
        compiler_params=pltpu.CompilerParams(
            dimension_semantics=("arbitrary",),
        ),
    )(encoding, enc_sq, wt, w_sq.reshape(1, k), embedding_weight)

    quantized_index = idx3.reshape(b)
    s = loss[0, 0] / (b * d)
    commitment_loss = s
    embedding_loss = s
    vq_loss = commitment_loss * _BETA + embedding_loss
    return (quantized_index, st, vq_loss, embedding_loss, commitment_loss)


# speedup vs baseline: 1.2738x; 1.2738x over previous
import functools

import jax
import jax.numpy as jnp
from jax import lax
from jax.experimental import pallas as pl
from jax.experimental.pallas import tpu as pltpu
from jax.experimental.pallas import tpu_sc as plsc

_BETA = 0.25
_BB = 128


def _vq_body(e_ref, esq_ref, wt_ref, wsq_ref, idx_ref, loss_ref):
    e = e_ref[...]
    mm = jnp.dot(e, wt_ref[...],
                 preferred_element_type=jnp.float32)
    d2 = (esq_ref[...] - 2.0 * mm) + wsq_ref[...]
    k = d2.shape[1]
    h = k // 2
    iota = lax.broadcasted_iota(jnp.int32, d2.shape, 1)
    dl, dr = d2[:, :h], d2[:, h:]
    vl = jnp.min(dl, axis=1, keepdims=True)
    vr = jnp.min(dr, axis=1, keepdims=True)
    il = jnp.min(jnp.where(dl == vl, iota[:, :h], k), axis=1)
    ir = jnp.min(jnp.where(dr == vr, iota[:, h:], k), axis=1)
    vl_bf = vl.astype(jnp.bfloat16).astype(jnp.float32)
    pickr = vr < vl_bf
    idx_ref[0, 0, :] = jnp.where(pickr[:, 0], ir, il)
    dsel = jnp.where(pickr[:, 0], vr[:, 0], vl[:, 0])
    @pl.when(pl.program_id(0) == 0)
    def _():
        loss_ref[...] = jnp.zeros((1, 1), jnp.float32)
    loss_ref[...] += jnp.sum(dsel).reshape(1, 1)


def _argmin_losses(encoding, embedding_weight):
    b, d = encoding.shape
    k = embedding_weight.shape[0]
    enc_sq = jnp.sum(encoding * encoding, axis=1, keepdims=True)
    w_sq = jnp.sum(embedding_weight * embedding_weight, axis=1)
    wt = embedding_weight.T
    g = b // _BB
    idx3, loss = pl.pallas_call(
        _vq_body,
        grid=(g,),
        in_specs=[
            pl.BlockSpec((_BB, d), lambda i: (i, 0)),
            pl.BlockSpec((_BB, 1), lambda i: (i, 0)),
            pl.BlockSpec((d, k), lambda i: (0, 0)),
            pl.BlockSpec((1, k), lambda i: (0, 0)),
        ],
        out_specs=[
            pl.BlockSpec((1, 1, _BB), lambda i: (i, 0, 0)),
            pl.BlockSpec((1, 1), lambda i: (0, 0)),
        ],
        out_shape=[
            jax.ShapeDtypeStruct((g, 1, _BB), jnp.int32),
            jax.ShapeDtypeStruct((1, 1), jnp.float32),
        ],
        compiler_params=pltpu.CompilerParams(
            dimension_semantics=("arbitrary",),
        ),
    )(encoding, enc_sq, wt, w_sq.reshape(1, k))
    return idx3.reshape(b), loss[0, 0]


def _make_sc_gather(b, k):
    info = plsc.get_sparse_core_info()
    nw = info.num_cores * info.num_subcores
    b_per_w = b // nw
    n_chunks = b_per_w // 128
    mesh = plsc.VectorSubcoreMesh(core_axis_name="c", subcore_axis_name="s")

    @functools.partial(
        pl.kernel, mesh=mesh,
        out_type=jax.ShapeDtypeStruct((b, 128), jnp.float32),
        scratch_types=[
            pltpu.VMEM((b_per_w,), jnp.int32),
            pltpu.VMEM((128, 128), jnp.float32),
            pltpu.SemaphoreType.DMA,
        ],
    )
    def gather_kernel(table_hbm, idx_hbm, out_hbm, idx_v, rows_v, sem):
        wid = lax.axis_index("s") * info.num_cores + lax.axis_index("c")
        base = wid * b_per_w
        pltpu.sync_copy(idx_hbm.at[pl.ds(base, b_per_w)], idx_v)

        def body(j, _):
            pltpu.async_copy(
                table_hbm.at[idx_v.at[pl.ds(j * 128, 128)]], rows_v, sem
            ).wait()
            pltpu.sync_copy(rows_v, out_hbm.at[pl.ds(base + j * 128, 128)])
            return ()

        lax.fori_loop(0, n_chunks, body, (), unroll=4)

    return gather_kernel


def kernel(encoding, embedding_weight):
    b, d = encoding.shape
    k = embedding_weight.shape[0]
    quantized_index, loss_sum = _argmin_losses(encoding, embedding_weight)
    table_p = jnp.pad(embedding_weight, ((0, 0), (0, 128 - d)))
    quantized_st = _make_sc_gather(b, k)(table_p, quantized_index)[:, :d]
    s = loss_sum / (b * d)
    commitment_loss = s
    embedding_loss = s
    vq_loss = commitment_loss * _BETA + embedding_loss
    return (quantized_index, quantized_st, vq_loss, embedding_loss, commitment_loss)


# speedup vs baseline: 1.5965x; 1.2533x over previous
import functools

import jax
import jax.numpy as jnp
from jax import lax
from jax.experimental import pallas as pl
from jax.experimental.pallas import tpu as pltpu
from jax.experimental.pallas import tpu_sc as plsc

_BETA = 0.25
_BB = 128


def _vq_body(e_ref, esq_ref, w2t_ref, wsq_ref, idx_ref, loss_ref):
    e = e_ref[...]
    bb = e.shape[0]
    mm2 = jnp.dot(e, w2t_ref[...],
                  preferred_element_type=jnp.float32)
    k = mm2.shape[1]
    ns = k // 128
    esq_b = jnp.broadcast_to(esq_ref[...], (bb, 128))
    wsq = wsq_ref[...]
    lane = lax.broadcasted_iota(jnp.int32, (bb, 128), 1)

    def half_argmin(s_lo, s_hi):
        av = jnp.full((bb, 128), jnp.inf, jnp.float32)
        ai = jnp.zeros((bb, 128), jnp.int32)
        for s in range(s_lo, s_hi):
            d2s = (esq_b - mm2[:, s * 128:(s + 1) * 128]) + jnp.broadcast_to(
                wsq[:, s * 128:(s + 1) * 128], (bb, 128))
            upd = d2s < av
            av = jnp.where(upd, d2s, av)
            ai = jnp.where(upd, s, ai)
        v = jnp.min(av, axis=1, keepdims=True)
        kfull = ai * 128 + lane
        i = jnp.min(jnp.where(av == v, kfull, k), axis=1)
        return v, i

    vl, il = half_argmin(0, ns // 2)
    vr, ir = half_argmin(ns // 2, ns)
    vl_bf = vl.astype(jnp.bfloat16).astype(jnp.float32)
    pickr = vr < vl_bf
    idx_ref[0, 0, :] = jnp.where(pickr[:, 0], ir, il)
    dsel = jnp.where(pickr[:, 0], vr[:, 0], vl[:, 0])
    @pl.when(pl.program_id(0) == 0)
    def _():
        loss_ref[...] = jnp.zeros((1, 1), jnp.float32)
    loss_ref[...] += jnp.sum(dsel).reshape(1, 1)


def _argmin_losses(encoding, embedding_weight):
    b, d = encoding.shape
    k = embedding_weight.shape[0]
    enc_sq = jnp.sum(encoding * encoding, axis=1, keepdims=True)
    w_sq = jnp.sum(embedding_weight * embedding_weight, axis=1)
    w2t = (2.0 * embedding_weight).T
    g = b // _BB
    idx3, loss = pl.pallas_call(
        _vq_body,
        grid=(g,),
        in_specs=[
            pl.BlockSpec((_BB, d), lambda i: (i, 0)),
            pl.BlockSpec((_BB, 1), lambda i: (i, 0)),
            pl.BlockSpec((d, k), lambda i: (0, 0)),
            pl.BlockSpec((1, k), lambda i: (0, 0)),
        ],
        out_specs=[
            pl.BlockSpec((1, 1, _BB), lambda i: (i, 0, 0)),
            pl.BlockSpec((1, 1), lambda i: (0, 0)),
        ],
        out_shape=[
            jax.ShapeDtypeStruct((g, 1, _BB), jnp.int32),
            jax.ShapeDtypeStruct((1, 1), jnp.float32),
        ],
        compiler_params=pltpu.CompilerParams(
            dimension_semantics=("arbitrary",),
        ),
    )(encoding, enc_sq, w2t, w_sq.reshape(1, k))
    return idx3.reshape(b), loss[0, 0]


def _make_sc_gather(b, k):
    info = plsc.get_sparse_core_info()
    nw = info.num_cores * info.num_subcores
    b_per_w = b // nw
    n_chunks = b_per_w // 128
    mesh = plsc.VectorSubcoreMesh(core_axis_name="c", subcore_axis_name="s")

    @functools.partial(
        pl.kernel, mesh=mesh,
        out_type=jax.ShapeDtypeStruct((b, 128), jnp.float32),
        scratch_types=[
            pltpu.VMEM((b_per_w,), jnp.int32),
            pltpu.VMEM((128, 128), jnp.float32),
            pltpu.SemaphoreType.DMA,
        ],
    )
    def gather_kernel(table_hbm, idx_hbm, out_hbm, idx_v, rows_v, sem):
        wid = lax.axis_index("s") * info.num_cores + lax.axis_index("c")
        base = wid * b_per_w
        pltpu.sync_copy(idx_hbm.at[pl.ds(base, b_per_w)], idx_v)

        def body(j, _):
            pltpu.async_copy(
                table_hbm.at[idx_v.at[pl.ds(j * 128, 128)]], rows_v, sem
            ).wait()
            pltpu.sync_copy(rows_v, out_hbm.at[pl.ds(base + j * 128, 128)])
            return ()

        lax.fori_loop(0, n_chunks, body, (), unroll=4)

    return gather_kernel


def kernel(encoding, embedding_weight):
    b, d = encoding.shape
    k = embedding_weight.shape[0]
    quantized_index, loss_sum = _argmin_losses(encoding, embedding_weight)
    table_p = jnp.pad(embedding_weight, ((0, 0), (0, 128 - d)))
    quantized_st = _make_sc_gather(b, k)(table_p, quantized_index)[:, :d]
    s = loss_sum / (b * d)
    commitment_loss = s
    embedding_loss = s
    vq_loss = commitment_loss * _BETA + embedding_loss
    return (quantized_index, quantized_st, vq_loss, embedding_loss, commitment_loss)
